# baseline (speedup 1.0000x reference)
import jax
import jax.numpy as jnp
from jax.experimental import pallas as pl

BATCH = 16384
E = 64
BT = 2048
HI = jax.lax.Precision.HIGHEST


def _body(x_ref, c_ref, r_ref, out_ref, idx_ref):
    xt = x_ref[...]
    c = c_ref[...]
    rt = r_ref[...]

    cn2 = jnp.sum(c * c, axis=1, keepdims=True)
    g = jax.lax.dot_general(c, xt, (((1,), (0,)), ((), ())),
                            precision=HI,
                            preferred_element_type=jnp.float32)
    s_rank = 0.5 * cn2 - g

    sub = jax.lax.broadcasted_iota(jnp.int32, (E, BT), 0)
    a1 = jnp.argmin(s_rank, axis=0)
    masked = jnp.where(sub == a1[None, :], jnp.inf, s_rank)
    a2 = jnp.argmin(masked, axis=0)

    h1 = (sub == a1[None, :]).astype(jnp.float32)
    h2 = (sub == a2[None, :]).astype(jnp.float32)

    c_hi = c.astype(jnp.bfloat16).astype(jnp.float32)
    c_r = c - c_hi
    c_mid = c_r.astype(jnp.bfloat16).astype(jnp.float32)
    c_lo = c_r - c_mid

    def _gather(h):
        p = [jax.lax.dot_general(cp, h, (((0,), (0,)), ((), ())),
                                 preferred_element_type=jnp.float32)
             for cp in (c_hi, c_mid, c_lo)]
        return (p[0] + p[1]) + p[2]

    c1t = _gather(h1)
    c2t = _gather(h2)

    dx1 = xt - c1t
    dx2 = xt - c2t
    s1 = jnp.sqrt(jnp.sum(dx1 * dx1, axis=0))
    s2 = jnp.sqrt(jnp.sum(dx2 * dx2, axis=0))

    pred = (s2 < s1) | ((s2 == s1) & (a2 < a1))
    best = jnp.where(pred, a2, a1)
    mind = jnp.where(pred, s2, s1)
    cselt = jnp.where(pred[None, :], c2t, c1t)
    r1 = jax.lax.dot_general(rt, h1, (((1,), (0,)), ((), ())),
                             precision=HI,
                             preferred_element_type=jnp.float32)
    r2 = jax.lax.dot_general(rt, h2, (((1,), (0,)), ((), ())),
                             precision=HI,
                             preferred_element_type=jnp.float32)
    rsel = jnp.where(pred, r2[0], r1[0])

    s = 0.1 * jnp.exp(-mind / (rsel + 1e-8))
    out_ref[...] = xt * (1.0 - s)[None, :] + cselt * s[None, :]
    idx_ref[...] = best.astype(jnp.int32)


def kernel(expert_activations, attractor_centers, attraction_radii):
    radii_row = attraction_radii.reshape(1, E)
    xt = jnp.transpose(expert_activations)
    outt, closest = pl.pallas_call(
        _body,
        grid=(BATCH // BT,),
        in_specs=[
            pl.BlockSpec((E, BT), lambda i: (0, i)),
            pl.BlockSpec((E, E), lambda i: (0, 0)),
            pl.BlockSpec((1, E), lambda i: (0, 0)),
        ],
        out_specs=[
            pl.BlockSpec((E, BT), lambda i: (0, i)),
            pl.BlockSpec((BT,), lambda i: (i,)),
        ],
        out_shape=[
            jax.ShapeDtypeStruct((E, BATCH), jnp.float32),
            jax.ShapeDtypeStruct((BATCH,), jnp.int32),
        ],
    )(xt, attractor_centers, radii_row)
    return jnp.transpose(outt), closest


# speedup vs baseline: 1.0604x; 1.0604x over previous
import jax
import jax.numpy as jnp
from jax.experimental import pallas as pl

BATCH = 16384
E = 64
BT = 8192
HI = jax.lax.Precision.HIGHEST


def _body(x_ref, c_ref, r_ref, out_ref, idx_ref):
    xt = x_ref[...]
    c = c_ref[...]
    rt = r_ref[...]

    cn2 = jnp.sum(c * c, axis=1, keepdims=True)
    g = jax.lax.dot_general(c, xt, (((1,), (0,)), ((), ())),
                            precision=HI,
                            preferred_element_type=jnp.float32)
    s_rank = 0.5 * cn2 - g

    sub = jax.lax.broadcasted_iota(jnp.int32, (E, BT), 0)
    a1 = jnp.argmin(s_rank, axis=0)
    masked = jnp.where(sub == a1[None, :], jnp.inf, s_rank)
    a2 = jnp.argmin(masked, axis=0)

    h1 = (sub == a1[None, :]).astype(jnp.float32)
    h2 = (sub == a2[None, :]).astype(jnp.float32)

    c_hi = c.astype(jnp.bfloat16).astype(jnp.float32)
    c_r = c - c_hi
    c_mid = c_r.astype(jnp.bfloat16).astype(jnp.float32)
    c_lo = c_r - c_mid

    def _gather(h):
        p = [jax.lax.dot_general(cp, h, (((0,), (0,)), ((), ())),
                                 preferred_element_type=jnp.float32)
             for cp in (c_hi, c_mid, c_lo)]
        return (p[0] + p[1]) + p[2]

    c1t = _gather(h1)
    c2t = _gather(h2)

    dx1 = xt - c1t
    dx2 = xt - c2t
    s1 = jnp.sqrt(jnp.sum(dx1 * dx1, axis=0))
    s2 = jnp.sqrt(jnp.sum(dx2 * dx2, axis=0))

    pred = (s2 < s1) | ((s2 == s1) & (a2 < a1))
    best = jnp.where(pred, a2, a1)
    mind = jnp.where(pred, s2, s1)
    cselt = jnp.where(pred[None, :], c2t, c1t)
    r1 = jax.lax.dot_general(rt, h1, (((1,), (0,)), ((), ())),
                             precision=HI,
                             preferred_element_type=jnp.float32)
    r2 = jax.lax.dot_general(rt, h2, (((1,), (0,)), ((), ())),
                             precision=HI,
                             preferred_element_type=jnp.float32)
    rsel = jnp.where(pred, r2[0], r1[0])

    s = 0.1 * jnp.exp(-mind / (rsel + 1e-8))
    out_ref[...] = xt * (1.0 - s)[None, :] + cselt * s[None, :]
    idx_ref[...] = best.astype(jnp.int32)


def kernel(expert_activations, attractor_centers, attraction_radii):
    radii_row = attraction_radii.reshape(1, E)
    xt = jnp.transpose(expert_activations)
    outt, closest = pl.pallas_call(
        _body,
        grid=(BATCH // BT,),
        in_specs=[
            pl.BlockSpec((E, BT), lambda i: (0, i)),
            pl.BlockSpec((E, E), lambda i: (0, 0)),
            pl.BlockSpec((1, E), lambda i: (0, 0)),
        ],
        out_specs=[
            pl.BlockSpec((E, BT), lambda i: (0, i)),
            pl.BlockSpec((BT,), lambda i: (i,)),
        ],
        out_shape=[
            jax.ShapeDtypeStruct((E, BATCH), jnp.float32),
            jax.ShapeDtypeStruct((BATCH,), jnp.int32),
        ],
    )(xt, attractor_centers, radii_row)
    return jnp.transpose(outt), closest


# speedup vs baseline: 1.1924x; 1.1245x over previous
import jax
import jax.numpy as jnp
from jax.experimental import pallas as pl

BATCH = 16384
E = 64
BT = 4096
HI = jax.lax.Precision.HIGHEST


def _body(x_ref, c_ref, r_ref, out_ref, idx_ref):
    xt = x_ref[...]
    c = c_ref[...]
    rt = r_ref[...]

    cn2 = jnp.sum(c * c, axis=1, keepdims=True)
    g = jax.lax.dot_general(c, xt, (((1,), (0,)), ((), ())),
                            precision=HI,
                            preferred_element_type=jnp.float32)
    s_rank = 0.5 * cn2 - g

    sub = jax.lax.broadcasted_iota(jnp.int32, (E, BT), 0)
    a1 = jnp.argmin(s_rank, axis=0)
    masked = jnp.where(sub == a1[None, :], jnp.inf, s_rank)
    a2 = jnp.argmin(masked, axis=0)

    h1 = (sub == a1[None, :]).astype(jnp.float32)
    h2 = (sub == a2[None, :]).astype(jnp.float32)

    c_hi = c.astype(jnp.bfloat16).astype(jnp.float32)
    c_r = c - c_hi
    c_mid = c_r.astype(jnp.bfloat16).astype(jnp.float32)
    c_lo = c_r - c_mid

    def _gather(h):
        p = [jax.lax.dot_general(cp, h, (((0,), (0,)), ((), ())),
                                 preferred_element_type=jnp.float32)
             for cp in (c_hi, c_mid, c_lo)]
        return (p[0] + p[1]) + p[2]

    c1t = _gather(h1)
    c2t = _gather(h2)

    dx1 = xt - c1t
    dx2 = xt - c2t
    s1 = jnp.sqrt(jnp.sum(dx1 * dx1, axis=0))
    s2 = jnp.sqrt(jnp.sum(dx2 * dx2, axis=0))

    pred = (s2 < s1) | ((s2 == s1) & (a2 < a1))
    best = jnp.where(pred, a2, a1)
    mind = jnp.where(pred, s2, s1)
    cselt = jnp.where(pred[None, :], c2t, c1t)
    r_hi = rt.astype(jnp.bfloat16).astype(jnp.float32)
    r_r = rt - r_hi
    r_mid = r_r.astype(jnp.bfloat16).astype(jnp.float32)
    r_lo = r_r - r_mid

    def _rgather(h):
        p = [jax.lax.dot_general(rp, h, (((1,), (0,)), ((), ())),
                                 preferred_element_type=jnp.float32)
             for rp in (r_hi, r_mid, r_lo)]
        return (p[0] + p[1]) + p[2]

    rsel = jnp.where(pred, _rgather(h2)[0], _rgather(h1)[0])

    s = 0.1 * jnp.exp(-mind / (rsel + 1e-8))
    out_ref[...] = xt * (1.0 - s)[None, :] + cselt * s[None, :]
    idx_ref[...] = best.astype(jnp.int32)


def kernel(expert_activations, attractor_centers, attraction_radii):
    radii_row = attraction_radii.reshape(1, E)
    xt = jnp.transpose(expert_activations)
    outt, closest = pl.pallas_call(
        _body,
        grid=(BATCH // BT,),
        in_specs=[
            pl.BlockSpec((E, BT), lambda i: (0, i)),
            pl.BlockSpec((E, E), lambda i: (0, 0)),
            pl.BlockSpec((1, E), lambda i: (0, 0)),
        ],
        out_specs=[
            pl.BlockSpec((E, BT), lambda i: (0, i)),
            pl.BlockSpec((BT,), lambda i: (i,)),
        ],
        out_shape=[
            jax.ShapeDtypeStruct((E, BATCH), jnp.float32),
            jax.ShapeDtypeStruct((BATCH,), jnp.int32),
        ],
    )(xt, attractor_centers, radii_row)
    return jnp.transpose(outt), closest
